# baseline (speedup 1.0000x reference)
import functools

import jax
import jax.numpy as jnp
from jax import lax
from jax.experimental import pallas as pl
from jax.experimental.pallas import tpu as pltpu
from jax.experimental.pallas import tpu_sc as plsc


_B, _L, _D = 16, 4096, 1024
_NC, _NS = 2, 16
_CH = 32
_SPLIT = 2432
_SC_ROWS = _L - _SPLIT
_HALF = _SC_ROWS // 2
_NCHUNK = _HALF // _CH
_VECS = _D // 16


def _sc_body(x_hbm, out_hbm, buf, acc, shared, sem0, sem1):
    c = lax.axis_index("c")
    s = lax.axis_index("s")
    b = c * (_NS // 2) + s // 2
    row0 = _SPLIT + (s % 2) * _HALF
    sems = (sem0, sem1)

    def _start(chunk, slot, sem):
        pltpu.async_copy(
            x_hbm.at[b, pl.ds(row0 + chunk * _CH, _CH), :],
            buf.at[slot],
            sem,
        )

    def _wait(slot, sem):
        pltpu.make_async_copy(
            x_hbm.at[b, pl.ds(row0, _CH), :], buf.at[slot], sem
        ).wait()

    zero = jnp.zeros((16,), jnp.float32)

    def _zero(j, _):
        acc[pl.ds(pl.multiple_of(j * 16, 16), 16)] = zero
        return ()

    lax.fori_loop(0, _VECS, _zero, ())

    _start(0, 0, sem0)
    _start(1, 1, sem1)

    def _accum(slot):
        def body(j, _):
            o = pl.multiple_of(j * 16, 16)
            rows = [buf[slot, r, pl.ds(o, 16)] for r in range(_CH)]
            while len(rows) > 1:
                rows = [rows[i] + rows[i + 1] for i in range(0, len(rows), 2)]
            acc[pl.ds(o, 16)] = acc[pl.ds(o, 16)] + rows[0]
            return ()

        lax.fori_loop(0, _VECS, body, ())

    def _main(g, _):
        for slot in (0, 1):
            chunk = 2 * g + slot
            _wait(slot, sems[slot])
            _accum(slot)
            nxt = chunk + 2

            @pl.when(nxt < _NCHUNK)
            def _():
                _start(nxt, slot, sems[slot])

        return ()

    lax.fori_loop(0, _NCHUNK // 2, _main, ())

    pltpu.sync_copy(acc, shared.at[s])
    plsc.subcore_barrier()

    @pl.when(s < _NS // 2)
    def _combine():
        pltpu.sync_copy(shared.at[2 * s], buf.at[0, 0])
        pltpu.sync_copy(shared.at[2 * s + 1], buf.at[0, 1])

        def body(j, _):
            o = pl.multiple_of(j * 16, 16)
            acc[pl.ds(o, 16)] = buf[0, 0, pl.ds(o, 16)] + buf[0, 1, pl.ds(o, 16)]
            return ()

        lax.fori_loop(0, _VECS, body, ())
        pltpu.sync_copy(acc, out_hbm.at[c * (_NS // 2) + s])


def _sc_sum(inputs):
    mesh = plsc.VectorSubcoreMesh(core_axis_name="c", subcore_axis_name="s")
    f = pl.kernel(
        _sc_body,
        mesh=mesh,
        out_type=jax.ShapeDtypeStruct((_B, _D), jnp.float32),
        scratch_types=[
            pltpu.VMEM((2, _CH, _D), jnp.float32),
            pltpu.VMEM((_D,), jnp.float32),
            pltpu.VMEM_SHARED((_NS, _D), jnp.float32),
            pltpu.SemaphoreType.DMA,
            pltpu.SemaphoreType.DMA,
        ],
    )
    return f(inputs)


_TC_CHUNK = 1216


def _tc_sum_kernel(x_ref, o_ref):
    @pl.when(pl.program_id(1) == 0)
    def _init():
        o_ref[...] = jnp.zeros_like(o_ref)

    o_ref[...] += jnp.sum(x_ref[...], axis=1, keepdims=True)


def _tc_sum(inputs):
    grid = (_B, _SPLIT // _TC_CHUNK)
    out = pl.pallas_call(
        _tc_sum_kernel,
        grid=grid,
        in_specs=[pl.BlockSpec((1, _TC_CHUNK, _D), lambda i, j: (i, j, 0))],
        out_specs=pl.BlockSpec((1, 1, _D), lambda i, j: (i, 0, 0)),
        out_shape=jax.ShapeDtypeStruct((_B, 1, _D), inputs.dtype),
        compiler_params=pltpu.CompilerParams(
            dimension_semantics=("parallel", "arbitrary"),
        ),
    )(inputs)
    return out.reshape(_B, _D)


def kernel(inputs):
    return _tc_sum(inputs) + _sc_sum(inputs)


# speedup vs baseline: 1.0098x; 1.0098x over previous
import jax
import jax.numpy as jnp
from jax import lax
from jax.experimental import pallas as pl
from jax.experimental.pallas import tpu as pltpu
from jax.experimental.pallas import tpu_sc as plsc


_B, _L, _D = 16, 4096, 1024
_NC, _NS = 2, 16
_CH = 32
_SPLIT = 2688
_SC_ROWS = _L - _SPLIT
_HALF = _SC_ROWS // 2
_NCHUNK = _HALF // _CH
_VECS = _D // 16


def _sc_body(x_hbm, out_hbm, buf, acc, shared, sem0, sem1):
    c = lax.axis_index("c")
    s = lax.axis_index("s")
    b = c * (_NS // 2) + s // 2
    row0 = _SPLIT + (s % 2) * _HALF
    sems = (sem0, sem1)

    def _start(chunk, slot, sem):
        pltpu.async_copy(
            x_hbm.at[b, pl.ds(row0 + chunk * _CH, _CH), :],
            buf.at[slot],
            sem,
        )

    def _wait(slot, sem):
        pltpu.make_async_copy(
            x_hbm.at[b, pl.ds(row0, _CH), :], buf.at[slot], sem
        ).wait()

    zero = jnp.zeros((16,), jnp.float32)

    def _zero(j, _):
        acc[pl.ds(pl.multiple_of(j * 16, 16), 16)] = zero
        return ()

    lax.fori_loop(0, _VECS, _zero, ())

    _start(0, 0, sem0)
    _start(1, 1, sem1)

    def _accum(slot):
        def body(j, _):
            o = pl.multiple_of(j * 16, 16)
            rows = [buf[slot, r, pl.ds(o, 16)] for r in range(_CH)]
            while len(rows) > 1:
                rows = [rows[i] + rows[i + 1] for i in range(0, len(rows), 2)]
            acc[pl.ds(o, 16)] = acc[pl.ds(o, 16)] + rows[0]
            return ()

        lax.fori_loop(0, _VECS, body, ())

    def _main(g, _):
        for slot in (0, 1):
            chunk = 2 * g + slot
            _wait(slot, sems[slot])
            _accum(slot)
            nxt = chunk + 2

            @pl.when(nxt < _NCHUNK)
            def _():
                _start(nxt, slot, sems[slot])

        return ()

    lax.fori_loop(0, _NCHUNK // 2, _main, ())

    pltpu.sync_copy(acc, shared.at[s])
    plsc.subcore_barrier()

    @pl.when(s < _NS // 2)
    def _combine():
        pltpu.sync_copy(shared.at[2 * s], buf.at[0, 0])
        pltpu.sync_copy(shared.at[2 * s + 1], buf.at[0, 1])

        def body(j, _):
            o = pl.multiple_of(j * 16, 16)
            acc[pl.ds(o, 16)] = buf[0, 0, pl.ds(o, 16)] + buf[0, 1, pl.ds(o, 16)]
            return ()

        lax.fori_loop(0, _VECS, body, ())
        pltpu.sync_copy(acc, out_hbm.at[c * (_NS // 2) + s])


def _sc_sum(inputs):
    mesh = plsc.VectorSubcoreMesh(core_axis_name="c", subcore_axis_name="s")
    f = pl.kernel(
        _sc_body,
        mesh=mesh,
        out_type=jax.ShapeDtypeStruct((_B, _D), jnp.float32),
        scratch_types=[
            pltpu.VMEM((2, _CH, _D), jnp.float32),
            pltpu.VMEM((_D,), jnp.float32),
            pltpu.VMEM_SHARED((_NS, _D), jnp.float32),
            pltpu.SemaphoreType.DMA,
            pltpu.SemaphoreType.DMA,
        ],
    )
    return f(inputs)


_TC_CHUNK = 1344


def _tc_sum_kernel(x_ref, o_ref):
    @pl.when(pl.program_id(1) == 0)
    def _init():
        o_ref[...] = jnp.zeros_like(o_ref)

    o_ref[...] += jnp.sum(x_ref[...], axis=1, keepdims=True)


def _tc_sum(inputs):
    grid = (_B, _SPLIT // _TC_CHUNK)
    out = pl.pallas_call(
        _tc_sum_kernel,
        grid=grid,
        in_specs=[pl.BlockSpec((1, _TC_CHUNK, _D), lambda i, j: (i, j, 0))],
        out_specs=pl.BlockSpec((1, 1, _D), lambda i, j: (i, 0, 0)),
        out_shape=jax.ShapeDtypeStruct((_B, 1, _D), inputs.dtype),
        compiler_params=pltpu.CompilerParams(
            dimension_semantics=("parallel", "arbitrary"),
        ),
    )(inputs)
    return out.reshape(_B, _D)


def kernel(inputs):
    return _tc_sum(inputs) + _sc_sum(inputs)


# speedup vs baseline: 1.0258x; 1.0158x over previous
import jax
import jax.numpy as jnp
from jax import lax
from jax.experimental import pallas as pl
from jax.experimental.pallas import tpu as pltpu
from jax.experimental.pallas import tpu_sc as plsc


_B, _L, _D = 16, 4096, 1024
_NC, _NS = 2, 16
_CH = 32
_SPLIT = 3584
_SC_ROWS = _L - _SPLIT
_HALF = _SC_ROWS // 2
_NCHUNK = _HALF // _CH
_VECS = _D // 16


def _sc_body(x_hbm, out_hbm, buf, acc, shared, sem0, sem1):
    c = lax.axis_index("c")
    s = lax.axis_index("s")
    b = c * (_NS // 2) + s // 2
    row0 = _SPLIT + (s % 2) * _HALF
    sems = (sem0, sem1)

    def _start(chunk, slot, sem):
        pltpu.async_copy(
            x_hbm.at[b, pl.ds(row0 + chunk * _CH, _CH), :],
            buf.at[slot],
            sem,
        )

    def _wait(slot, sem):
        pltpu.make_async_copy(
            x_hbm.at[b, pl.ds(row0, _CH), :], buf.at[slot], sem
        ).wait()

    zero = jnp.zeros((16,), jnp.float32)

    def _zero(j, _):
        acc[pl.ds(pl.multiple_of(j * 16, 16), 16)] = zero
        return ()

    lax.fori_loop(0, _VECS, _zero, ())

    _start(0, 0, sem0)
    _start(1, 1, sem1)

    def _accum(slot):
        def body(j, _):
            o = pl.multiple_of(j * 16, 16)
            rows = [buf[slot, r, pl.ds(o, 16)] for r in range(_CH)]
            while len(rows) > 1:
                rows = [rows[i] + rows[i + 1] for i in range(0, len(rows), 2)]
            acc[pl.ds(o, 16)] = acc[pl.ds(o, 16)] + rows[0]
            return ()

        lax.fori_loop(0, _VECS, body, ())

    def _main(g, _):
        for slot in (0, 1):
            chunk = 2 * g + slot
            _wait(slot, sems[slot])
            _accum(slot)
            nxt = chunk + 2

            @pl.when(nxt < _NCHUNK)
            def _():
                _start(nxt, slot, sems[slot])

        return ()

    lax.fori_loop(0, _NCHUNK // 2, _main, ())

    pltpu.sync_copy(acc, shared.at[s])
    plsc.subcore_barrier()

    @pl.when(s < _NS // 2)
    def _combine():
        pltpu.sync_copy(shared.at[2 * s], buf.at[0, 0])
        pltpu.sync_copy(shared.at[2 * s + 1], buf.at[0, 1])

        def body(j, _):
            o = pl.multiple_of(j * 16, 16)
            acc[pl.ds(o, 16)] = buf[0, 0, pl.ds(o, 16)] + buf[0, 1, pl.ds(o, 16)]
            return ()

        lax.fori_loop(0, _VECS, body, ())
        pltpu.sync_copy(acc, out_hbm.at[c * (_NS // 2) + s])


def _sc_sum(inputs):
    mesh = plsc.VectorSubcoreMesh(core_axis_name="c", subcore_axis_name="s")
    f = pl.kernel(
        _sc_body,
        mesh=mesh,
        out_type=jax.ShapeDtypeStruct((_B, _D), jnp.float32),
        scratch_types=[
            pltpu.VMEM((2, _CH, _D), jnp.float32),
            pltpu.VMEM((_D,), jnp.float32),
            pltpu.VMEM_SHARED((_NS, _D), jnp.float32),
            pltpu.SemaphoreType.DMA,
            pltpu.SemaphoreType.DMA,
        ],
    )
    return f(inputs)


_TC_CHUNK = 1792


def _tc_sum_kernel(x_ref, o_ref):
    @pl.when(pl.program_id(1) == 0)
    def _init():
        o_ref[...] = jnp.zeros_like(o_ref)

    o_ref[...] += jnp.sum(x_ref[...], axis=1, keepdims=True)


def _tc_sum(inputs):
    grid = (_B, _SPLIT // _TC_CHUNK)
    out = pl.pallas_call(
        _tc_sum_kernel,
        grid=grid,
        in_specs=[pl.BlockSpec((1, _TC_CHUNK, _D), lambda i, j: (i, j, 0))],
        out_specs=pl.BlockSpec((1, 1, _D), lambda i, j: (i, 0, 0)),
        out_shape=jax.ShapeDtypeStruct((_B, 1, _D), inputs.dtype),
        compiler_params=pltpu.CompilerParams(
            dimension_semantics=("parallel", "arbitrary"),
        ),
    )(inputs)
    return out.reshape(_B, _D)


def kernel(inputs):
    return _tc_sum(inputs) + _sc_sum(inputs)


# speedup vs baseline: 1.0291x; 1.0032x over previous
import jax
import jax.numpy as jnp
from jax import lax
from jax.experimental import pallas as pl
from jax.experimental.pallas import tpu as pltpu
from jax.experimental.pallas import tpu_sc as plsc


_B, _L, _D = 16, 4096, 1024
_NC, _NS = 2, 16
_CH = 32
_SPLIT = 3840
_SC_ROWS = _L - _SPLIT
_HALF = _SC_ROWS // 2
_NCHUNK = _HALF // _CH
_VECS = _D // 16


def _sc_body(x_hbm, out_hbm, buf, acc, shared, sem0, sem1):
    c = lax.axis_index("c")
    s = lax.axis_index("s")
    b = c * (_NS // 2) + s // 2
    row0 = _SPLIT + (s % 2) * _HALF
    sems = (sem0, sem1)

    def _start(chunk, slot, sem):
        pltpu.async_copy(
            x_hbm.at[b, pl.ds(row0 + chunk * _CH, _CH), :],
            buf.at[slot],
            sem,
        )

    def _wait(slot, sem):
        pltpu.make_async_copy(
            x_hbm.at[b, pl.ds(row0, _CH), :], buf.at[slot], sem
        ).wait()

    zero = jnp.zeros((16,), jnp.float32)

    def _zero(j, _):
        acc[pl.ds(pl.multiple_of(j * 16, 16), 16)] = zero
        return ()

    lax.fori_loop(0, _VECS, _zero, ())

    _start(0, 0, sem0)
    _start(1, 1, sem1)

    def _accum(slot):
        def body(j, _):
            o = pl.multiple_of(j * 16, 16)
            rows = [buf[slot, r, pl.ds(o, 16)] for r in range(_CH)]
            while len(rows) > 1:
                rows = [rows[i] + rows[i + 1] for i in range(0, len(rows), 2)]
            acc[pl.ds(o, 16)] = acc[pl.ds(o, 16)] + rows[0]
            return ()

        lax.fori_loop(0, _VECS, body, ())

    def _main(g, _):
        for slot in (0, 1):
            chunk = 2 * g + slot
            _wait(slot, sems[slot])
            _accum(slot)
            nxt = chunk + 2

            @pl.when(nxt < _NCHUNK)
            def _():
                _start(nxt, slot, sems[slot])

        return ()

    lax.fori_loop(0, _NCHUNK // 2, _main, ())

    pltpu.sync_copy(acc, shared.at[s])
    plsc.subcore_barrier()

    @pl.when(s < _NS // 2)
    def _combine():
        pltpu.sync_copy(shared.at[2 * s], buf.at[0, 0])
        pltpu.sync_copy(shared.at[2 * s + 1], buf.at[0, 1])

        def body(j, _):
            o = pl.multiple_of(j * 16, 16)
            acc[pl.ds(o, 16)] = buf[0, 0, pl.ds(o, 16)] + buf[0, 1, pl.ds(o, 16)]
            return ()

        lax.fori_loop(0, _VECS, body, ())
        pltpu.sync_copy(acc, out_hbm.at[c * (_NS // 2) + s])


def _sc_sum(inputs):
    mesh = plsc.VectorSubcoreMesh(core_axis_name="c", subcore_axis_name="s")
    f = pl.kernel(
        _sc_body,
        mesh=mesh,
        out_type=jax.ShapeDtypeStruct((_B, _D), jnp.float32),
        scratch_types=[
            pltpu.VMEM((2, _CH, _D), jnp.float32),
            pltpu.VMEM((_D,), jnp.float32),
            pltpu.VMEM_SHARED((_NS, _D), jnp.float32),
            pltpu.SemaphoreType.DMA,
            pltpu.SemaphoreType.DMA,
        ],
    )
    return f(inputs)


_TC_CHUNK = 1920


def _tc_sum_kernel(x_ref, o_ref):
    @pl.when(pl.program_id(1) == 0)
    def _init():
        o_ref[...] = jnp.zeros_like(o_ref)

    o_ref[...] += jnp.sum(x_ref[...], axis=1, keepdims=True)


def _tc_sum(inputs):
    grid = (_B, _SPLIT // _TC_CHUNK)
    out = pl.pallas_call(
        _tc_sum_kernel,
        grid=grid,
        in_specs=[pl.BlockSpec((1, _TC_CHUNK, _D), lambda i, j: (i, j, 0))],
        out_specs=pl.BlockSpec((1, 1, _D), lambda i, j: (i, 0, 0)),
        out_shape=jax.ShapeDtypeStruct((_B, 1, _D), inputs.dtype),
        compiler_params=pltpu.CompilerParams(
            dimension_semantics=("parallel", "arbitrary"),
        ),
    )(inputs)
    return out.reshape(_B, _D)


def kernel(inputs):
    return _tc_sum(inputs) + _sc_sum(inputs)
